# Initial kernel scaffold; baseline (speedup 1.0000x reference)
import jax
import jax.numpy as jnp
from jax.experimental import pallas as pl


def kernel(query, keys, values, W1, W2, Wg1, Wg2):
    raise NotImplementedError("write your pallas kernel here")


# speedup vs baseline: 1.9704x; 1.9704x over previous
import functools
import jax
import jax.numpy as jnp
from jax import lax
from jax.experimental import pallas as pl
from jax.experimental.pallas import tpu as pltpu
from jax.experimental.pallas import tpu_sc as plsc

B, Q, S, D = 4, 128, 8192, 2048
BLK = 2048
NBLK = S // BLK
WBLK = NBLK - 1
NSEL = 64
BS = 16
NROWS = B * NSEL * BS
_VMEM_LIM = pltpu.CompilerParams(vmem_limit_bytes=64 * 1024 * 1024)


def _kpass_body(q_ref, k_ref, sk_ref, p_ref):
    i = pl.program_id(1)
    k = k_ref[0]
    sk_ref[0] = jnp.sum(k, axis=0, keepdims=True)

    @pl.when(i == WBLK)
    def _():
        s = lax.dot_general(q_ref[0], k, (((1,), (1,)), ((), ())))
        m = jnp.max(s, axis=1, keepdims=True)
        p = jnp.exp(s - m)
        p_ref[0] = p / jnp.sum(p, axis=1, keepdims=True)


def _run_kpass(query, keys):
    return pl.pallas_call(
        _kpass_body,
        grid=(B, NBLK),
        in_specs=[
            pl.BlockSpec((1, Q, D), lambda b, i: (b, 0, 0)),
            pl.BlockSpec((1, BLK, D), lambda b, i: (b, i, 0)),
        ],
        out_specs=[
            pl.BlockSpec((1, 1, D), lambda b, i: (b * NBLK + i, 0, 0)),
            pl.BlockSpec((1, Q, BLK), lambda b, i: (b, 0, 0)),
        ],
        out_shape=[
            jax.ShapeDtypeStruct((B * NBLK, 1, D), jnp.float32),
            jax.ShapeDtypeStruct((B, Q, BLK), jnp.float32),
        ],
        compiler_params=_VMEM_LIM,
    )(query, keys)


def _vpass_body(p_ref, v_ref, sv_ref, ow_ref):
    i = pl.program_id(1)
    v = v_ref[0]
    sv_ref[0] = jnp.sum(v, axis=0, keepdims=True)

    @pl.when(i == WBLK)
    def _():
        ow_ref[0] = jnp.dot(p_ref[0], v)


def _run_vpass(pwin, values):
    return pl.pallas_call(
        _vpass_body,
        grid=(B, NBLK),
        in_specs=[
            pl.BlockSpec((1, Q, BLK), lambda b, i: (b, 0, 0)),
            pl.BlockSpec((1, BLK, D), lambda b, i: (b, i, 0)),
        ],
        out_specs=[
            pl.BlockSpec((1, 1, D), lambda b, i: (b * NBLK + i, 0, 0)),
            pl.BlockSpec((1, Q, D), lambda b, i: (b, 0, 0)),
        ],
        out_shape=[
            jax.ShapeDtypeStruct((B * NBLK, 1, D), jnp.float32),
            jax.ShapeDtypeStruct((B, Q, D), jnp.float32),
        ],
        compiler_params=_VMEM_LIM,
    )(pwin, values)


def _small_mm_body(x_ref, w_ref, o_ref, *, relu):
    r = lax.dot_general(x_ref[...], w_ref[...], (((1,), (1,)), ((), ())))
    o_ref[...] = jnp.maximum(r, 0.0) if relu else r


def _small_mm(x, w, relu):
    return pl.pallas_call(
        functools.partial(_small_mm_body, relu=relu),
        out_shape=jax.ShapeDtypeStruct((x.shape[0], w.shape[0]), jnp.float32),
        compiler_params=_VMEM_LIM,
    )(x, w)


def _scores_body(ck_ref, cv_ref, q_ref, oc_ref, tok_ref):
    dn = (((1,), (1,)), ((), ()))
    ck = ck_ref[...]
    cv = cv_ref[...]
    NSC = Q * NBLK

    qq1 = lax.broadcasted_iota(jnp.int32, (Q, Q), 0)
    qq2 = lax.broadcasted_iota(jnp.int32, (Q, Q), 1)
    r64 = lax.broadcasted_iota(jnp.int32, (NSEL, Q), 0)
    q64 = lax.broadcasted_iota(jnp.int32, (NSEL, Q), 1)

    for b in range(B):
        qb = q_ref[b]
        ckb = ck[b * NBLK:(b + 1) * NBLK]
        cvb = cv[b * NBLK:(b + 1) * NBLK]
        logits = lax.dot_general(qb, ckb, dn)
        mx = jnp.max(logits, axis=1, keepdims=True)
        e = jnp.exp(logits - mx)
        scores = e / jnp.sum(e, axis=1, keepdims=True)
        oc_ref[b] = jnp.dot(scores, cvb)

        st = jnp.transpose(scores)
        rank_rows = []
        for c in range(NBLK):
            row_c = st[c:c + 1, :]
            r = jnp.zeros((1, Q), jnp.int32)
            for cp in range(NBLK):
                col_cp = scores[:, cp:cp + 1]
                gt = col_cp > row_c
                if cp < c:
                    tlt = qq1 <= qq2
                else:
                    tlt = qq1 < qq2
                r = r + jnp.sum(
                    jnp.where(gt | ((col_cp == row_c) & tlt), 1, 0),
                    axis=0, keepdims=True)
            rank_rows.append(r)

        idx64 = jnp.zeros((NSEL, 1), jnp.int32)
        for c in range(NBLK):
            oh = jnp.broadcast_to(rank_rows[c], (NSEL, Q)) == r64
            idx64 = idx64 + jnp.sum(
                jnp.where(oh, q64 * NBLK + c, 0), axis=1, keepdims=True)
        t16 = lax.broadcasted_iota(jnp.int32, (1, BS), 1)
        tok = idx64 * BS + t16 + b * S
        tok_ref[pl.ds(b * NSEL, NSEL), :] = tok


def _run_scores(sums_k, sums_v, query, W1, W2):
    means = jnp.concatenate(
        [sums_k.reshape(B * NBLK, D), sums_v.reshape(B * NBLK, D)],
        axis=0) * (1.0 / 2048.0)
    h = _small_mm(means, W1, relu=True)
    ckcv = _small_mm(h, W2, relu=False)
    oc, tok = pl.pallas_call(
        _scores_body,
        out_shape=[
            jax.ShapeDtypeStruct((B, Q, D), jnp.float32),
            jax.ShapeDtypeStruct((B * NSEL, BS), jnp.int32),
        ],
        compiler_params=_VMEM_LIM,
    )(ckcv[:B * NBLK], ckcv[B * NBLK:], query)
    return oc, tok


def _gate_body(q_ref, wg1_ref, wg2_ref, g_ref):
    q = q_ref[...].reshape(B * Q, D)
    h = jnp.maximum(
        lax.dot_general(q, wg1_ref[...], (((1,), (1,)), ((), ()))), 0.0)
    g = lax.dot_general(h, wg2_ref[...], (((1,), (1,)), ((), ())))
    g_ref[...] = jax.nn.sigmoid(g).reshape(B, Q, 8)


def _run_gate(query, Wg1, Wg2):
    wg2p = jnp.zeros((8, D), jnp.float32).at[:3].set(Wg2)
    return pl.pallas_call(
        _gate_body,
        out_shape=jax.ShapeDtypeStruct((B, Q, 8), jnp.float32),
        compiler_params=_VMEM_LIM,
    )(query, Wg1, wg2p)


_GCH = 32


def _sc_gather(keys2d, values2d, tok):
    mesh = plsc.VectorSubcoreMesh(core_axis_name="c", subcore_axis_name="s")
    info = plsc.get_sparse_core_info()
    nw = info.num_cores * info.num_subcores
    rows_per_w = NROWS // nw

    @functools.partial(
        pl.kernel, mesh=mesh,
        out_type=[
            jax.ShapeDtypeStruct((NROWS, D), jnp.float32),
            jax.ShapeDtypeStruct((NROWS, D), jnp.float32),
        ],
        scratch_types=[
            pltpu.VMEM((_GCH,), jnp.int32),
            pltpu.VMEM((_GCH, D), jnp.float32),
            pltpu.SemaphoreType.DMA,
        ],
    )
    def k(keys_hbm, values_hbm, idx_hbm, sk_hbm, sv_hbm, idx_v, rows_v, sem):
        wid = lax.axis_index("s") * info.num_cores + lax.axis_index("c")
        for i in range(rows_per_w // _GCH):
            base = wid * rows_per_w + i * _GCH
            pltpu.sync_copy(idx_hbm.at[pl.ds(base, _GCH)], idx_v)
            pltpu.async_copy(keys_hbm.at[idx_v], rows_v, sem).wait()
            pltpu.sync_copy(rows_v, sk_hbm.at[pl.ds(base, _GCH)])
            pltpu.async_copy(values_hbm.at[idx_v], rows_v, sem).wait()
            pltpu.sync_copy(rows_v, sv_hbm.at[pl.ds(base, _GCH)])

    return k(keys2d, values2d, tok.reshape(NROWS))


def _combine_body(q_ref, sk_ref, sv_ref, oc_ref, ow_ref, g_ref, out_ref):
    q = q_ref[0]
    s = lax.dot_general(q, sk_ref[0], (((1,), (1,)), ((), ())))
    m = jnp.max(s, axis=1, keepdims=True)
    p = jnp.exp(s - m)
    osel = jnp.dot(p, sv_ref[0]) / jnp.sum(p, axis=1, keepdims=True)
    g = g_ref[0]
    out_ref[0] = (g[:, 0:1] * oc_ref[0] + g[:, 1:2] * osel
                  + g[:, 2:3] * ow_ref[0])


def _run_combine(query, sk, sv, oc, ow, gates):
    return pl.pallas_call(
        _combine_body,
        grid=(B,),
        in_specs=[
            pl.BlockSpec((1, Q, D), lambda b: (b, 0, 0)),
            pl.BlockSpec((1, NSEL * BS, D), lambda b: (b, 0, 0)),
            pl.BlockSpec((1, NSEL * BS, D), lambda b: (b, 0, 0)),
            pl.BlockSpec((1, Q, D), lambda b: (b, 0, 0)),
            pl.BlockSpec((1, Q, D), lambda b: (b, 0, 0)),
            pl.BlockSpec((1, Q, 8), lambda b: (b, 0, 0)),
        ],
        out_specs=pl.BlockSpec((1, Q, D), lambda b: (b, 0, 0)),
        out_shape=jax.ShapeDtypeStruct((B, Q, D), jnp.float32),
        compiler_params=_VMEM_LIM,
    )(query, sk, sv, oc, ow, gates)


def kernel(query, keys, values, W1, W2, Wg1, Wg2):
    sums_k, pwin = _run_kpass(query, keys)
    sums_v, ow = _run_vpass(pwin, values)
    oc, tok = _run_scores(sums_k, sums_v, query, W1, W2)
    gates = _run_gate(query, Wg1, Wg2)
    sk2d, sv2d = _sc_gather(keys.reshape(B * S, D), values.reshape(B * S, D),
                            tok)
    sk = sk2d.reshape(B, NSEL * BS, D)
    sv = sv2d.reshape(B, NSEL * BS, D)
    return _run_combine(query, sk, sv, oc, ow, gates)


# speedup vs baseline: 1.9953x; 1.0126x over previous
import functools
import jax
import jax.numpy as jnp
from jax import lax
from jax.experimental import pallas as pl
from jax.experimental.pallas import tpu as pltpu
from jax.experimental.pallas import tpu_sc as plsc

B, Q, S, D = 4, 128, 8192, 2048
BLK = 2048
NBLK = S // BLK
WBLK = NBLK - 1
NSEL = 64
BS = 16
NROWS = B * NSEL * BS
_VMEM_LIM = pltpu.CompilerParams(vmem_limit_bytes=64 * 1024 * 1024)


def _kpass_body(q_ref, k_ref, sk_ref, p_ref):
    i = pl.program_id(1)
    k = k_ref[0]
    sk_ref[0] = jnp.sum(k, axis=0, keepdims=True)

    @pl.when(i == WBLK)
    def _():
        s = lax.dot_general(q_ref[0], k, (((1,), (1,)), ((), ())))
        m = jnp.max(s, axis=1, keepdims=True)
        p = jnp.exp(s - m)
        p_ref[0] = p / jnp.sum(p, axis=1, keepdims=True)


def _run_kpass(query, keys):
    return pl.pallas_call(
        _kpass_body,
        grid=(B, NBLK),
        in_specs=[
            pl.BlockSpec((1, Q, D), lambda b, i: (b, 0, 0)),
            pl.BlockSpec((1, BLK, D), lambda b, i: (b, i, 0)),
        ],
        out_specs=[
            pl.BlockSpec((1, 1, D), lambda b, i: (b * NBLK + i, 0, 0)),
            pl.BlockSpec((1, Q, BLK), lambda b, i: (b, 0, 0)),
        ],
        out_shape=[
            jax.ShapeDtypeStruct((B * NBLK, 1, D), jnp.float32),
            jax.ShapeDtypeStruct((B, Q, BLK), jnp.float32),
        ],
        compiler_params=_VMEM_LIM,
    )(query, keys)


def _vpass_body(p_ref, v_ref, sv_ref, ow_ref):
    i = pl.program_id(1)
    v = v_ref[0]
    sv_ref[0] = jnp.sum(v, axis=0, keepdims=True)

    @pl.when(i == WBLK)
    def _():
        ow_ref[0] = jnp.dot(p_ref[0], v)


def _run_vpass(pwin, values):
    return pl.pallas_call(
        _vpass_body,
        grid=(B, NBLK),
        in_specs=[
            pl.BlockSpec((1, Q, BLK), lambda b, i: (b, 0, 0)),
            pl.BlockSpec((1, BLK, D), lambda b, i: (b, i, 0)),
        ],
        out_specs=[
            pl.BlockSpec((1, 1, D), lambda b, i: (b * NBLK + i, 0, 0)),
            pl.BlockSpec((1, Q, D), lambda b, i: (b, 0, 0)),
        ],
        out_shape=[
            jax.ShapeDtypeStruct((B * NBLK, 1, D), jnp.float32),
            jax.ShapeDtypeStruct((B, Q, D), jnp.float32),
        ],
        compiler_params=_VMEM_LIM,
    )(pwin, values)


def _small_mm_body(x_ref, w_ref, o_ref, *, relu):
    r = lax.dot_general(x_ref[...], w_ref[...], (((1,), (1,)), ((), ())))
    o_ref[...] = jnp.maximum(r, 0.0) if relu else r


def _small_mm(x, w, relu):
    return pl.pallas_call(
        functools.partial(_small_mm_body, relu=relu),
        out_shape=jax.ShapeDtypeStruct((x.shape[0], w.shape[0]), jnp.float32),
        compiler_params=_VMEM_LIM,
    )(x, w)


def _scores_body(ck_ref, cv_ref, q_ref, oc_ref, tok_ref):
    dn = (((1,), (1,)), ((), ()))
    ck = ck_ref[...]
    cv = cv_ref[...]
    NSC = Q * NBLK

    qq1 = lax.broadcasted_iota(jnp.int32, (Q, Q), 0)
    qq2 = lax.broadcasted_iota(jnp.int32, (Q, Q), 1)
    r64 = lax.broadcasted_iota(jnp.int32, (NSEL, Q), 0)
    q64 = lax.broadcasted_iota(jnp.int32, (NSEL, Q), 1)

    for b in range(B):
        qb = q_ref[b]
        ckb = ck[b * NBLK:(b + 1) * NBLK]
        cvb = cv[b * NBLK:(b + 1) * NBLK]
        logits = lax.dot_general(qb, ckb, dn)
        mx = jnp.max(logits, axis=1, keepdims=True)
        e = jnp.exp(logits - mx)
        scores = e / jnp.sum(e, axis=1, keepdims=True)
        oc_ref[b] = jnp.dot(scores, cvb)

        st = jnp.transpose(scores)
        rank_rows = []
        for c in range(NBLK):
            row_c = st[c:c + 1, :]
            r = jnp.zeros((1, Q), jnp.int32)
            for cp in range(NBLK):
                col_cp = scores[:, cp:cp + 1]
                gt = col_cp > row_c
                if cp < c:
                    tlt = qq1 <= qq2
                else:
                    tlt = qq1 < qq2
                r = r + jnp.sum(
                    jnp.where(gt | ((col_cp == row_c) & tlt), 1, 0),
                    axis=0, keepdims=True)
            rank_rows.append(r)

        idx64 = jnp.zeros((NSEL, 1), jnp.int32)
        for c in range(NBLK):
            oh = jnp.broadcast_to(rank_rows[c], (NSEL, Q)) == r64
            idx64 = idx64 + jnp.sum(
                jnp.where(oh, q64 * NBLK + c, 0), axis=1, keepdims=True)
        t16 = lax.broadcasted_iota(jnp.int32, (1, BS), 1)
        tok = idx64 * BS + t16 + b * S
        tok_ref[pl.ds(b * NSEL, NSEL), :] = tok


def _run_scores(sums_k, sums_v, query, W1, W2):
    means = jnp.concatenate(
        [sums_k.reshape(B * NBLK, D), sums_v.reshape(B * NBLK, D)],
        axis=0) * (1.0 / 2048.0)
    h = _small_mm(means, W1, relu=True)
    ckcv = _small_mm(h, W2, relu=False)
    oc, tok = pl.pallas_call(
        _scores_body,
        out_shape=[
            jax.ShapeDtypeStruct((B, Q, D), jnp.float32),
            jax.ShapeDtypeStruct((B * NSEL, BS), jnp.int32),
        ],
        compiler_params=_VMEM_LIM,
    )(ckcv[:B * NBLK], ckcv[B * NBLK:], query)
    return oc, tok


def _gate_body(q_ref, wg1_ref, wg2_ref, g_ref):
    q = q_ref[...].reshape(B * Q, D)
    h = jnp.maximum(
        lax.dot_general(q, wg1_ref[...], (((1,), (1,)), ((), ()))), 0.0)
    g = lax.dot_general(h, wg2_ref[...], (((1,), (1,)), ((), ())))
    g_ref[...] = jax.nn.sigmoid(g).reshape(B, Q, 8)


def _run_gate(query, Wg1, Wg2):
    wg2p = jnp.zeros((8, D), jnp.float32).at[:3].set(Wg2)
    return pl.pallas_call(
        _gate_body,
        out_shape=jax.ShapeDtypeStruct((B, Q, 8), jnp.float32),
        compiler_params=_VMEM_LIM,
    )(query, Wg1, wg2p)


_GCH = 8


def _sc_gather(keys2d, values2d, tok):
    mesh = plsc.VectorSubcoreMesh(core_axis_name="c", subcore_axis_name="s")
    info = plsc.get_sparse_core_info()
    nw = info.num_cores * info.num_subcores
    rows_per_w = NROWS // nw
    nch = rows_per_w // _GCH

    @functools.partial(
        pl.kernel, mesh=mesh,
        out_type=[
            jax.ShapeDtypeStruct((NROWS, D), jnp.float32),
            jax.ShapeDtypeStruct((NROWS, D), jnp.float32),
        ],
        scratch_types=[
            pltpu.VMEM((2, _GCH), jnp.int32),
            pltpu.VMEM((2, _GCH, D), jnp.float32),
            pltpu.VMEM((2, _GCH, D), jnp.float32),
            pltpu.SemaphoreType.DMA,
            pltpu.SemaphoreType.DMA,
            pltpu.SemaphoreType.DMA,
            pltpu.SemaphoreType.DMA,
            pltpu.SemaphoreType.DMA,
            pltpu.SemaphoreType.DMA,
            pltpu.SemaphoreType.DMA,
            pltpu.SemaphoreType.DMA,
        ],
    )
    def k(keys_hbm, values_hbm, idx_hbm, sk_hbm, sv_hbm,
          idx_v, rk, rv, gk0, gk1, gv0, gv1, wk0, wk1, wv0, wv1):
        wid = lax.axis_index("s") * info.num_cores + lax.axis_index("c")
        gsem = (gk0, gk1)
        vsem = (gv0, gv1)
        wksem = (wk0, wk1)
        wvsem = (wv0, wv1)
        gathers = [None] * nch
        writes = [None] * nch
        for i in range(nch):
            sl = i % 2
            base = wid * rows_per_w + i * _GCH
            if i >= 2:
                wa, wb = writes[i - 2]
                wa.wait()
                wb.wait()
            pltpu.sync_copy(idx_hbm.at[pl.ds(base, _GCH)], idx_v.at[sl])
            ga = pltpu.async_copy(keys_hbm.at[idx_v.at[sl]], rk.at[sl],
                                  gsem[sl])
            gb = pltpu.async_copy(values_hbm.at[idx_v.at[sl]], rv.at[sl],
                                  vsem[sl])
            gathers[i] = (ga, gb)
            if i >= 1:
                pb = (i - 1) % 2
                pbase = wid * rows_per_w + (i - 1) * _GCH
                pa, pvb = gathers[i - 1]
                pa.wait()
                wa = pltpu.async_copy(rk.at[pb], sk_hbm.at[pl.ds(pbase, _GCH)],
                                      wksem[pb])
                pvb.wait()
                wb = pltpu.async_copy(rv.at[pb], sv_hbm.at[pl.ds(pbase, _GCH)],
                                      wvsem[pb])
                writes[i - 1] = (wa, wb)
        last = nch - 1
        sl = last % 2
        base = wid * rows_per_w + last * _GCH
        pa, pvb = gathers[last]
        pa.wait()
        wa = pltpu.async_copy(rk.at[sl], sk_hbm.at[pl.ds(base, _GCH)],
                              wksem[sl])
        pvb.wait()
        wb = pltpu.async_copy(rv.at[sl], sv_hbm.at[pl.ds(base, _GCH)],
                              wvsem[sl])
        wa.wait()
        wb.wait()
        w2 = writes[last - 1]
        w2[0].wait()
        w2[1].wait()

    return k(keys2d, values2d, tok.reshape(NROWS))


def _combine_body(q_ref, sk_ref, sv_ref, oc_ref, ow_ref, g_ref, out_ref):
    q = q_ref[0]
    s = lax.dot_general(q, sk_ref[0], (((1,), (1,)), ((), ())))
    m = jnp.max(s, axis=1, keepdims=True)
    p = jnp.exp(s - m)
    osel = jnp.dot(p, sv_ref[0]) / jnp.sum(p, axis=1, keepdims=True)
    g = g_ref[0]
    out_ref[0] = (g[:, 0:1] * oc_ref[0] + g[:, 1:2] * osel
                  + g[:, 2:3] * ow_ref[0])


def _run_combine(query, sk, sv, oc, ow, gates):
    return pl.pallas_call(
        _combine_body,
        grid=(B,),
        in_specs=[
            pl.BlockSpec((1, Q, D), lambda b: (b, 0, 0)),
            pl.BlockSpec((1, NSEL * BS, D), lambda b: (b, 0, 0)),
            pl.BlockSpec((1, NSEL * BS, D), lambda b: (b, 0, 0)),
            pl.BlockSpec((1, Q, D), lambda b: (b, 0, 0)),
            pl.BlockSpec((1, Q, D), lambda b: (b, 0, 0)),
            pl.BlockSpec((1, Q, 8), lambda b: (b, 0, 0)),
        ],
        out_specs=pl.BlockSpec((1, Q, D), lambda b: (b, 0, 0)),
        out_shape=jax.ShapeDtypeStruct((B, Q, D), jnp.float32),
        compiler_params=_VMEM_LIM,
    )(query, sk, sv, oc, ow, gates)


def kernel(query, keys, values, W1, W2, Wg1, Wg2):
    sums_k, pwin = _run_kpass(query, keys)
    sums_v, ow = _run_vpass(pwin, values)
    oc, tok = _run_scores(sums_k, sums_v, query, W1, W2)
    sk2d, sv2d = _sc_gather(keys.reshape(B * S, D), values.reshape(B * S, D),
                            tok)
    gates = _run_gate(query, Wg1, Wg2)
    sk = sk2d.reshape(B, NSEL * BS, D)
    sv = sv2d.reshape(B, NSEL * BS, D)
    return _run_combine(query, sk, sv, oc, ow, gates)


# speedup vs baseline: 2.0465x; 1.0257x over previous
import functools
import jax
import jax.numpy as jnp
from jax import lax
from jax.experimental import pallas as pl
from jax.experimental.pallas import tpu as pltpu
from jax.experimental.pallas import tpu_sc as plsc

B, Q, S, D = 4, 128, 8192, 2048
BLK = 2048
NBLK = S // BLK
WBLK = NBLK - 1
NSEL = 64
BS = 16
NROWS = B * NSEL * BS
_VMEM_LIM = pltpu.CompilerParams(vmem_limit_bytes=64 * 1024 * 1024)


def _kpass_body(q_ref, k_ref, sk_ref, p_ref):
    i = pl.program_id(1)
    k = k_ref[0]
    sk_ref[0] = jnp.sum(k, axis=0, keepdims=True)

    @pl.when(i == WBLK)
    def _():
        s = lax.dot_general(q_ref[0], k, (((1,), (1,)), ((), ())))
        m = jnp.max(s, axis=1, keepdims=True)
        p = jnp.exp(s - m)
        p_ref[0] = p / jnp.sum(p, axis=1, keepdims=True)


def _run_kpass(query, keys):
    return pl.pallas_call(
        _kpass_body,
        grid=(B, NBLK),
        in_specs=[
            pl.BlockSpec((1, Q, D), lambda b, i: (b, 0, 0)),
            pl.BlockSpec((1, BLK, D), lambda b, i: (b, i, 0)),
        ],
        out_specs=[
            pl.BlockSpec((1, 1, D), lambda b, i: (b * NBLK + i, 0, 0)),
            pl.BlockSpec((1, Q, BLK), lambda b, i: (b, 0, 0)),
        ],
        out_shape=[
            jax.ShapeDtypeStruct((B * NBLK, 1, D), jnp.float32),
            jax.ShapeDtypeStruct((B, Q, BLK), jnp.float32),
        ],
        compiler_params=_VMEM_LIM,
    )(query, keys)


def _vpass_body(p_ref, v_ref, sv_ref, ow_ref):
    i = pl.program_id(1)
    v = v_ref[0]
    sv_ref[0] = jnp.sum(v, axis=0, keepdims=True)

    @pl.when(i == WBLK)
    def _():
        ow_ref[0] = jnp.dot(p_ref[0], v)


def _run_vpass(pwin, values):
    return pl.pallas_call(
        _vpass_body,
        grid=(B, NBLK),
        in_specs=[
            pl.BlockSpec((1, Q, BLK), lambda b, i: (b, 0, 0)),
            pl.BlockSpec((1, BLK, D), lambda b, i: (b, i, 0)),
        ],
        out_specs=[
            pl.BlockSpec((1, 1, D), lambda b, i: (b * NBLK + i, 0, 0)),
            pl.BlockSpec((1, Q, D), lambda b, i: (b, 0, 0)),
        ],
        out_shape=[
            jax.ShapeDtypeStruct((B * NBLK, 1, D), jnp.float32),
            jax.ShapeDtypeStruct((B, Q, D), jnp.float32),
        ],
        compiler_params=_VMEM_LIM,
    )(pwin, values)


def _scores_body(means_ref, w1_ref, w2_ref, q_ref, oc_ref, tok_ref):
    dn = (((1,), (1,)), ((), ()))
    h = jnp.maximum(lax.dot_general(means_ref[...], w1_ref[...], dn), 0.0)
    ckcv = lax.dot_general(h, w2_ref[...], dn)
    ck = ckcv[:B * NBLK]
    cv = ckcv[B * NBLK:]
    NSC = Q * NBLK

    qq1 = lax.broadcasted_iota(jnp.int32, (Q, Q), 0)
    qq2 = lax.broadcasted_iota(jnp.int32, (Q, Q), 1)
    r64 = lax.broadcasted_iota(jnp.int32, (NSEL, Q), 0)
    q64 = lax.broadcasted_iota(jnp.int32, (NSEL, Q), 1)

    for b in range(B):
        qb = q_ref[b]
        ckb = ck[b * NBLK:(b + 1) * NBLK]
        cvb = cv[b * NBLK:(b + 1) * NBLK]
        logits = lax.dot_general(qb, ckb, dn)
        mx = jnp.max(logits, axis=1, keepdims=True)
        e = jnp.exp(logits - mx)
        scores = e / jnp.sum(e, axis=1, keepdims=True)
        oc_ref[b] = jnp.dot(scores, cvb)

        st = jnp.transpose(scores)
        rank_rows = []
        for c in range(NBLK):
            row_c = st[c:c + 1, :]
            r = jnp.zeros((1, Q), jnp.int32)
            for cp in range(NBLK):
                col_cp = scores[:, cp:cp + 1]
                gt = col_cp > row_c
                if cp < c:
                    tlt = qq1 <= qq2
                else:
                    tlt = qq1 < qq2
                r = r + jnp.sum(
                    jnp.where(gt | ((col_cp == row_c) & tlt), 1, 0),
                    axis=0, keepdims=True)
            rank_rows.append(r)

        idx64 = jnp.zeros((NSEL, 1), jnp.int32)
        for c in range(NBLK):
            oh = jnp.broadcast_to(rank_rows[c], (NSEL, Q)) == r64
            idx64 = idx64 + jnp.sum(
                jnp.where(oh, q64 * NBLK + c, 0), axis=1, keepdims=True)
        t16 = lax.broadcasted_iota(jnp.int32, (1, BS), 1)
        tok = idx64 * BS + t16 + b * S
        tok_ref[pl.ds(b * NSEL, NSEL), :] = tok


def _run_scores(sums_k, sums_v, query, W1, W2):
    means = jnp.concatenate(
        [sums_k.reshape(B * NBLK, D), sums_v.reshape(B * NBLK, D)],
        axis=0) * (1.0 / 2048.0)
    oc, tok = pl.pallas_call(
        _scores_body,
        out_shape=[
            jax.ShapeDtypeStruct((B, Q, D), jnp.float32),
            jax.ShapeDtypeStruct((B * NSEL, BS), jnp.int32),
        ],
        compiler_params=_VMEM_LIM,
    )(means, W1, W2, query)
    return oc, tok


def _gate_body(q_ref, wg1_ref, wg2_ref, g_ref):
    q = q_ref[...].reshape(B * Q, D)
    h = jnp.maximum(
        lax.dot_general(q, wg1_ref[...], (((1,), (1,)), ((), ()))), 0.0)
    g = lax.dot_general(h, wg2_ref[...], (((1,), (1,)), ((), ())))
    g_ref[...] = jax.nn.sigmoid(g).reshape(B, Q, 8)


def _run_gate(query, Wg1, Wg2):
    wg2p = jnp.zeros((8, D), jnp.float32).at[:3].set(Wg2)
    return pl.pallas_call(
        _gate_body,
        out_shape=jax.ShapeDtypeStruct((B, Q, 8), jnp.float32),
        compiler_params=_VMEM_LIM,
    )(query, Wg1, wg2p)


_GCH = 16
_NSLOT = 3


def _sc_gather(keys2d, values2d, tok):
    mesh = plsc.VectorSubcoreMesh(core_axis_name="c", subcore_axis_name="s")
    info = plsc.get_sparse_core_info()
    nw = info.num_cores * info.num_subcores
    rows_per_w = NROWS // nw
    nch = rows_per_w // _GCH

    @functools.partial(
        pl.kernel, mesh=mesh,
        out_type=[
            jax.ShapeDtypeStruct((NROWS, D), jnp.float32),
            jax.ShapeDtypeStruct((NROWS, D), jnp.float32),
        ],
        scratch_types=[
            pltpu.VMEM((rows_per_w,), jnp.int32),
            pltpu.VMEM((_NSLOT, _GCH, D), jnp.float32),
            pltpu.SemaphoreType.DMA,
            pltpu.SemaphoreType.DMA,
            pltpu.SemaphoreType.DMA,
            pltpu.SemaphoreType.DMA,
            pltpu.SemaphoreType.DMA,
            pltpu.SemaphoreType.DMA,
        ],
    )
    def k(keys_hbm, values_hbm, idx_hbm, sk_hbm, sv_hbm,
          idx_v, rbuf, g0, g1, g2, w0, w1, w2):
        wid = lax.axis_index("s") * info.num_cores + lax.axis_index("c")
        gsem = (g0, g1, g2)
        wsem = (w0, w1, w2)
        base_w = wid * rows_per_w
        pltpu.sync_copy(idx_hbm.at[pl.ds(base_w, rows_per_w)], idx_v)
        nt = 2 * nch
        gathers = [None] * nt
        writes = [None] * nt
        for t in range(nt):
            sl = t % _NSLOT
            ch = t // 2
            src = keys_hbm if t % 2 == 0 else values_hbm
            if t >= _NSLOT:
                writes[t - _NSLOT].wait()
            gathers[t] = pltpu.async_copy(
                src.at[idx_v.at[pl.ds(ch * _GCH, _GCH)]], rbuf.at[sl],
                gsem[sl])
            if t >= 1:
                tp = t - 1
                slp = tp % _NSLOT
                chp = tp // 2
                dst = sk_hbm if tp % 2 == 0 else sv_hbm
                gathers[tp].wait()
                writes[tp] = pltpu.async_copy(
                    rbuf.at[slp], dst.at[pl.ds(base_w + chp * _GCH, _GCH)],
                    wsem[slp])
        tl = nt - 1
        gathers[tl].wait()
        writes[tl] = pltpu.async_copy(
            rbuf.at[tl % _NSLOT],
            sv_hbm.at[pl.ds(base_w + (tl // 2) * _GCH, _GCH)],
            wsem[tl % _NSLOT])
        for t in range(nt - _NSLOT, nt):
            writes[t].wait()

    return k(keys2d, values2d, tok.reshape(NROWS))


def _combine_body(q_ref, sk_ref, sv_ref, oc_ref, ow_ref, g_ref, out_ref):
    q = q_ref[0]
    s = lax.dot_general(q, sk_ref[0], (((1,), (1,)), ((), ())))
    m = jnp.max(s, axis=1, keepdims=True)
    p = jnp.exp(s - m)
    osel = jnp.dot(p, sv_ref[0]) / jnp.sum(p, axis=1, keepdims=True)
    g = g_ref[0]
    out_ref[0] = (g[:, 0:1] * oc_ref[0] + g[:, 1:2] * osel
                  + g[:, 2:3] * ow_ref[0])


def _run_combine(query, sk, sv, oc, ow, gates):
    return pl.pallas_call(
        _combine_body,
        grid=(B,),
        in_specs=[
            pl.BlockSpec((1, Q, D), lambda b: (b, 0, 0)),
            pl.BlockSpec((1, NSEL * BS, D), lambda b: (b, 0, 0)),
            pl.BlockSpec((1, NSEL * BS, D), lambda b: (b, 0, 0)),
            pl.BlockSpec((1, Q, D), lambda b: (b, 0, 0)),
            pl.BlockSpec((1, Q, D), lambda b: (b, 0, 0)),
            pl.BlockSpec((1, Q, 8), lambda b: (b, 0, 0)),
        ],
        out_specs=pl.BlockSpec((1, Q, D), lambda b: (b, 0, 0)),
        out_shape=jax.ShapeDtypeStruct((B, Q, D), jnp.float32),
        compiler_params=_VMEM_LIM,
    )(query, sk, sv, oc, ow, gates)


def kernel(query, keys, values, W1, W2, Wg1, Wg2):
    sums_k, pwin = _run_kpass(query, keys)
    sums_v, ow = _run_vpass(pwin, values)
    oc, tok = _run_scores(sums_k, sums_v, query, W1, W2)
    sk2d, sv2d = _sc_gather(keys.reshape(B * S, D), values.reshape(B * S, D),
                            tok)
    gates = _run_gate(query, Wg1, Wg2)
    sk = sk2d.reshape(B, NSEL * BS, D)
    sv = sv2d.reshape(B, NSEL * BS, D)
    return _run_combine(query, sk, sv, oc, ow, gates)


# speedup vs baseline: 2.0471x; 1.0003x over previous
import functools
import jax
import jax.numpy as jnp
from jax import lax
from jax.experimental import pallas as pl
from jax.experimental.pallas import tpu as pltpu
from jax.experimental.pallas import tpu_sc as plsc

B, Q, S, D = 4, 128, 8192, 2048
BLK = 2048
NBLK = S // BLK
WBLK = NBLK - 1
NSEL = 64
BS = 16
NROWS = B * NSEL * BS
_VMEM_LIM = pltpu.CompilerParams(vmem_limit_bytes=64 * 1024 * 1024)


def _kpass_body(q_ref, k_ref, sk_ref, p_ref):
    i = pl.program_id(1)
    k = k_ref[0]
    sk_ref[0] = jnp.sum(k, axis=0, keepdims=True)

    @pl.when(i == WBLK)
    def _():
        s = lax.dot_general(q_ref[0], k, (((1,), (1,)), ((), ())))
        m = jnp.max(s, axis=1, keepdims=True)
        p = jnp.exp(s - m)
        p_ref[0] = p / jnp.sum(p, axis=1, keepdims=True)


def _run_kpass(query, keys):
    return pl.pallas_call(
        _kpass_body,
        grid=(B, NBLK),
        in_specs=[
            pl.BlockSpec((1, Q, D), lambda b, i: (b, 0, 0)),
            pl.BlockSpec((1, BLK, D), lambda b, i: (b, i, 0)),
        ],
        out_specs=[
            pl.BlockSpec((1, 1, D), lambda b, i: (b * NBLK + i, 0, 0)),
            pl.BlockSpec((1, Q, BLK), lambda b, i: (b, 0, 0)),
        ],
        out_shape=[
            jax.ShapeDtypeStruct((B * NBLK, 1, D), jnp.float32),
            jax.ShapeDtypeStruct((B, Q, BLK), jnp.float32),
        ],
        compiler_params=_VMEM_LIM,
    )(query, keys)


def _vpass_body(p_ref, v_ref, sv_ref, ow_ref):
    i = pl.program_id(1)
    v = v_ref[0]
    sv_ref[0] = jnp.sum(v, axis=0, keepdims=True)

    @pl.when(i == WBLK)
    def _():
        ow_ref[0] = jnp.dot(p_ref[0], v)


def _run_vpass(pwin, values):
    return pl.pallas_call(
        _vpass_body,
        grid=(B, NBLK),
        in_specs=[
            pl.BlockSpec((1, Q, BLK), lambda b, i: (b, 0, 0)),
            pl.BlockSpec((1, BLK, D), lambda b, i: (b, i, 0)),
        ],
        out_specs=[
            pl.BlockSpec((1, 1, D), lambda b, i: (b * NBLK + i, 0, 0)),
            pl.BlockSpec((1, Q, D), lambda b, i: (b, 0, 0)),
        ],
        out_shape=[
            jax.ShapeDtypeStruct((B * NBLK, 1, D), jnp.float32),
            jax.ShapeDtypeStruct((B, Q, D), jnp.float32),
        ],
        compiler_params=_VMEM_LIM,
    )(pwin, values)


def _scores_body(means_ref, w1_ref, w2_ref, q_ref, oc_ref, tok_ref):
    dn = (((1,), (1,)), ((), ()))
    h = jnp.maximum(lax.dot_general(means_ref[...], w1_ref[...], dn), 0.0)
    ckcv = lax.dot_general(h, w2_ref[...], dn)
    ck = ckcv[:B * NBLK]
    cv = ckcv[B * NBLK:]
    NSC = Q * NBLK

    qq1 = lax.broadcasted_iota(jnp.int32, (Q, Q), 0)
    qq2 = lax.broadcasted_iota(jnp.int32, (Q, Q), 1)
    r64 = lax.broadcasted_iota(jnp.int32, (NSEL, Q), 0)
    q64 = lax.broadcasted_iota(jnp.int32, (NSEL, Q), 1)

    for b in range(B):
        qb = q_ref[b]
        ckb = ck[b * NBLK:(b + 1) * NBLK]
        cvb = cv[b * NBLK:(b + 1) * NBLK]
        logits = lax.dot_general(qb, ckb, dn)
        mx = jnp.max(logits, axis=1, keepdims=True)
        e = jnp.exp(logits - mx)
        scores = e / jnp.sum(e, axis=1, keepdims=True)
        oc_ref[b] = jnp.dot(scores, cvb)

        st = jnp.transpose(scores)
        rank_rows = []
        for c in range(NBLK):
            row_c = st[c:c + 1, :]
            r = jnp.zeros((1, Q), jnp.int32)
            for cp in range(NBLK):
                col_cp = scores[:, cp:cp + 1]
                gt = col_cp > row_c
                if cp < c:
                    tlt = qq1 <= qq2
                else:
                    tlt = qq1 < qq2
                r = r + jnp.sum(
                    jnp.where(gt | ((col_cp == row_c) & tlt), 1, 0),
                    axis=0, keepdims=True)
            rank_rows.append(r)

        idx64 = jnp.zeros((NSEL, 1), jnp.int32)
        for c in range(NBLK):
            oh = jnp.broadcast_to(rank_rows[c], (NSEL, Q)) == r64
            idx64 = idx64 + jnp.sum(
                jnp.where(oh, q64 * NBLK + c, 0), axis=1, keepdims=True)
        t16 = lax.broadcasted_iota(jnp.int32, (1, BS), 1)
        tok = idx64 * BS + t16 + b * S
        tok_ref[pl.ds(b * NSEL, NSEL), :] = tok


def _run_scores(sums_k, sums_v, query, W1, W2):
    means = jnp.concatenate(
        [sums_k.reshape(B * NBLK, D), sums_v.reshape(B * NBLK, D)],
        axis=0) * (1.0 / 2048.0)
    oc, tok = pl.pallas_call(
        _scores_body,
        out_shape=[
            jax.ShapeDtypeStruct((B, Q, D), jnp.float32),
            jax.ShapeDtypeStruct((B * NSEL, BS), jnp.int32),
        ],
        compiler_params=_VMEM_LIM,
    )(means, W1, W2, query)
    return oc, tok


def _gate_body(q_ref, wg1_ref, wg2_ref, g_ref):
    q = q_ref[...].reshape(B * Q, D)
    h = jnp.maximum(
        lax.dot_general(q, wg1_ref[...], (((1,), (1,)), ((), ()))), 0.0)
    g = lax.dot_general(h, wg2_ref[...], (((1,), (1,)), ((), ())))
    g_ref[...] = jax.nn.sigmoid(g).reshape(B, Q, 8)


def _run_gate(query, Wg1, Wg2):
    wg2p = jnp.zeros((8, D), jnp.float32).at[:3].set(Wg2)
    return pl.pallas_call(
        _gate_body,
        out_shape=jax.ShapeDtypeStruct((B, Q, 8), jnp.float32),
        compiler_params=_VMEM_LIM,
    )(query, Wg1, wg2p)


_GCH = 8
_NSLOT = 6


def _sc_gather(keys2d, values2d, tok):
    mesh = plsc.VectorSubcoreMesh(core_axis_name="c", subcore_axis_name="s")
    info = plsc.get_sparse_core_info()
    nw = info.num_cores * info.num_subcores
    rows_per_w = NROWS // nw
    nch = rows_per_w // _GCH

    @functools.partial(
        pl.kernel, mesh=mesh,
        out_type=[
            jax.ShapeDtypeStruct((NROWS, D), jnp.float32),
            jax.ShapeDtypeStruct((NROWS, D), jnp.float32),
        ],
        scratch_types=[
            pltpu.VMEM((rows_per_w,), jnp.int32),
            pltpu.VMEM((_NSLOT, _GCH, D), jnp.float32),
        ] + [pltpu.SemaphoreType.DMA] * (2 * _NSLOT),
    )
    def k(keys_hbm, values_hbm, idx_hbm, sk_hbm, sv_hbm,
          idx_v, rbuf, *sems):
        wid = lax.axis_index("s") * info.num_cores + lax.axis_index("c")
        gsem = sems[:_NSLOT]
        wsem = sems[_NSLOT:]
        base_w = wid * rows_per_w
        pltpu.sync_copy(idx_hbm.at[pl.ds(base_w, rows_per_w)], idx_v)
        nt = 2 * nch
        gathers = [None] * nt
        writes = [None] * nt
        for t in range(nt):
            sl = t % _NSLOT
            ch = t // 2
            src = keys_hbm if t % 2 == 0 else values_hbm
            if t >= _NSLOT:
                writes[t - _NSLOT].wait()
            gathers[t] = pltpu.async_copy(
                src.at[idx_v.at[pl.ds(ch * _GCH, _GCH)]], rbuf.at[sl],
                gsem[sl])
            if t >= 1:
                tp = t - 1
                slp = tp % _NSLOT
                chp = tp // 2
                dst = sk_hbm if tp % 2 == 0 else sv_hbm
                gathers[tp].wait()
                writes[tp] = pltpu.async_copy(
                    rbuf.at[slp], dst.at[pl.ds(base_w + chp * _GCH, _GCH)],
                    wsem[slp])
        tl = nt - 1
        gathers[tl].wait()
        writes[tl] = pltpu.async_copy(
            rbuf.at[tl % _NSLOT],
            sv_hbm.at[pl.ds(base_w + (tl // 2) * _GCH, _GCH)],
            wsem[tl % _NSLOT])
        for t in range(nt - _NSLOT, nt):
            writes[t].wait()

    return k(keys2d, values2d, tok.reshape(NROWS))


def _combine_body(q_ref, sk_ref, sv_ref, oc_ref, ow_ref, g_ref, out_ref):
    q = q_ref[0]
    s = lax.dot_general(q, sk_ref[0], (((1,), (1,)), ((), ())))
    m = jnp.max(s, axis=1, keepdims=True)
    p = jnp.exp(s - m)
    osel = jnp.dot(p, sv_ref[0]) / jnp.sum(p, axis=1, keepdims=True)
    g = g_ref[0]
    out_ref[0] = (g[:, 0:1] * oc_ref[0] + g[:, 1:2] * osel
                  + g[:, 2:3] * ow_ref[0])


def _run_combine(query, sk, sv, oc, ow, gates):
    return pl.pallas_call(
        _combine_body,
        grid=(B,),
        in_specs=[
            pl.BlockSpec((1, Q, D), lambda b: (b, 0, 0)),
            pl.BlockSpec((1, NSEL * BS, D), lambda b: (b, 0, 0)),
            pl.BlockSpec((1, NSEL * BS, D), lambda b: (b, 0, 0)),
            pl.BlockSpec((1, Q, D), lambda b: (b, 0, 0)),
            pl.BlockSpec((1, Q, D), lambda b: (b, 0, 0)),
            pl.BlockSpec((1, Q, 8), lambda b: (b, 0, 0)),
        ],
        out_specs=pl.BlockSpec((1, Q, D), lambda b: (b, 0, 0)),
        out_shape=jax.ShapeDtypeStruct((B, Q, D), jnp.float32),
        compiler_params=_VMEM_LIM,
    )(query, sk, sv, oc, ow, gates)


def kernel(query, keys, values, W1, W2, Wg1, Wg2):
    sums_k, pwin = _run_kpass(query, keys)
    sums_v, ow = _run_vpass(pwin, values)
    oc, tok = _run_scores(sums_k, sums_v, query, W1, W2)
    sk2d, sv2d = _sc_gather(keys.reshape(B * S, D), values.reshape(B * S, D),
                            tok)
    gates = _run_gate(query, Wg1, Wg2)
    sk = sk2d.reshape(B, NSEL * BS, D)
    sv = sv2d.reshape(B, NSEL * BS, D)
    return _run_combine(query, sk, sv, oc, ow, gates)
